# baseline (speedup 1.0000x reference)
import functools

import jax
import jax.numpy as jnp
from jax import lax
from jax.experimental import pallas as pl
from jax.experimental.pallas import tpu as pltpu
from jax.experimental.pallas import tpu_sc as plsc

N = 10000
D = 128
H = 64
E2 = 160000
EB = 128
NC = 2
NS = 16
TILE_E = E2 // NS
NBLK = TILE_E // EB
REM = TILE_E % EB
DST_OFF = 2 * E2
ACC_R = 5376
P1 = 5104
DUMMY = P1
NPH = 2


def _eattr_body(ea_ref, wet_ref, lo_ref, hi_ref):
    r = jnp.dot(ea_ref[0], wet_ref[0], preferred_element_type=jnp.float32)
    lo_ref[...] = r[:, :H]
    hi_ref[...] = r[:, H:]


def _eattr_call(conv, ea3, WeT):
    Be = 8000
    nb = E2 // Be
    link_off = (conv % 2) * nb
    return pl.pallas_call(
        _eattr_body,
        grid=(nb,),
        in_specs=[
            pl.BlockSpec((1, Be, 16), lambda e: (0, link_off + e, 0)),
            pl.BlockSpec((1, 16, D), lambda e: (conv, 0, 0)),
        ],
        out_specs=[
            pl.BlockSpec((Be, H), lambda e: (e, 0)),
            pl.BlockSpec((Be, H), lambda e: (e, 0)),
        ],
        out_shape=[
            jax.ShapeDtypeStruct((E2, H), jnp.float32),
            jax.ShapeDtypeStruct((E2, H), jnp.float32),
        ],
        compiler_params=pltpu.CompilerParams(
            dimension_semantics=("arbitrary",)),
    )(ea3, WeT)


def _edge_body(ei_hbm, x_hbm, e0lo, e0hi, e1lo, e1hi,
               a0lo, a0hi, a1lo, a1hi,
               idx_s0, idx_s1, idx_d0, idx_d1, xj0, xj1, ea0, ea1, ov0, ov1,
               idx_s16, idx_d16, xj16, ea16, ov16,
               acc, g0, g1, ss):
    cid = lax.axis_index("c")
    sid = lax.axis_index("s")
    fin = xj0
    res = ea0

    def _stripe(ph, fn):
        full = jnp.where(ph == 0, 39, 38)
        for q in range(3):
            cix = q * NS + sid

            @pl.when(cix < full)
            def _():
                fn(pl.multiple_of(cix * 128, 128), 128)

        @pl.when((ph == 0) & (sid == 39 % NS))
        def _():
            fn(4992, 112)

        @pl.when((ph == 1) & (sid == 38 % NS))
        def _():
            fn(4864, 32)

    def _zero_ov():
        def _zrow(r, c):
            for k in range(8):
                ov0[r, pl.ds(k * 16, 16)] = jnp.zeros((16,), jnp.float32)
            return c
        lax.fori_loop(0, EB, _zrow, 0)

    def _compute(xjr, ear, ovr, nrows, xoff):
        def _row(r, c2):
            for k in range(4):
                xv = xjr[r, pl.ds(xoff + k * 16, 16)]
                ev = ear[r, pl.ds(k * 16, 16)]
                m = jnp.maximum(xv + ev, 0.0) + 1e-7
                ex = jnp.exp(m)
                ovr[r, pl.ds(H + k * 16, 16)] = ex
                ovr[r, pl.ds(k * 16, 16)] = m * ex
            return c2
        lax.fori_loop(0, nrows, _row, 0)

    def _remap(idxr, nrows, pbase, pend):
        def _rm(t, c2):
            dv = idxr[pl.ds(t * 16, 16)]
            ok = (dv >= pbase) & (dv < pend)
            idxr[pl.ds(t * 16, 16)] = jnp.where(
                ok, dv - pbase, jnp.full((16,), DUMMY, jnp.int32))
            return c2
        lax.fori_loop(0, nrows // 16, _rm, 0)

    def _scan(link_off, elo_hbm, ehi_hbm, pbase, pend):
        ebase = sid * TILE_E

        def _issue(blk, idx_s, xj, ea, gsem):
            bi = pl.multiple_of(link_off + ebase + blk * EB, 8)
            be = pl.multiple_of(ebase + blk * EB, 8)
            pltpu.sync_copy(ei_hbm.at[pl.ds(bi, EB)], idx_s)
            pltpu.async_copy(x_hbm.at[idx_s], xj, gsem)

            @pl.when(cid == 0)
            def _():
                pltpu.async_copy(elo_hbm.at[pl.ds(be, EB)], ea, gsem)

            @pl.when(cid == 1)
            def _():
                pltpu.async_copy(ehi_hbm.at[pl.ds(be, EB)], ea, gsem)

        def _wait_in(xj, ea, gsem):
            pltpu.make_async_copy(x_hbm.at[pl.ds(0, EB)], xj, gsem).wait()
            pltpu.make_async_copy(elo_hbm.at[pl.ds(0, EB)], ea, gsem).wait()

        def _wait_sc(ov):
            pltpu.make_async_copy(ov, acc.at[pl.ds(0, EB)], ss).wait()

        def _load_d(blk, idx_d):
            bi = pl.multiple_of(link_off + ebase + blk * EB, 8)
            pltpu.sync_copy(ei_hbm.at[pl.ds(DST_OFF + bi, EB)], idx_d)

        def _comp(xj, ea, ov):
            @pl.when(cid == 0)
            def _():
                _compute(xj, ea, ov, EB, 0)

            @pl.when(cid == 1)
            def _():
                _compute(xj, ea, ov, EB, H)

        def _blk(j, c):
            bi = pl.multiple_of(link_off + ebase + j * EB, 8)
            be = pl.multiple_of(ebase + j * EB, 8)
            pltpu.sync_copy(ei_hbm.at[pl.ds(bi, EB)], idx_s0)
            pltpu.sync_copy(ei_hbm.at[pl.ds(DST_OFF + bi, EB)], idx_d0)
            pltpu.async_copy(x_hbm.at[idx_s0], xj0, g0).wait()

            @pl.when(cid == 0)
            def _():
                pltpu.sync_copy(elo_hbm.at[pl.ds(be, EB)], ea0)

            @pl.when(cid == 1)
            def _():
                pltpu.sync_copy(ehi_hbm.at[pl.ds(be, EB)], ea0)

            _remap(idx_d0, EB, pbase, pend)
            _comp(xj0, ea0, ov0)
            pltpu.sync_copy(ov0, acc.at[idx_d0], add=True)
            return c
        lax.fori_loop(0, NBLK, _blk, 0)

        re_ = pl.multiple_of(ebase + NBLK * EB, 8)
        ri = pl.multiple_of(link_off + ebase + NBLK * EB, 8)
        pltpu.sync_copy(ei_hbm.at[pl.ds(ri, REM)], idx_s16)
        pltpu.sync_copy(ei_hbm.at[pl.ds(DST_OFF + ri, REM)], idx_d16)
        pltpu.async_copy(x_hbm.at[idx_s16], xj16, ss).wait()

        @pl.when(cid == 0)
        def _():
            pltpu.sync_copy(elo_hbm.at[pl.ds(re_, REM)], ea16)

        @pl.when(cid == 1)
        def _():
            pltpu.sync_copy(ehi_hbm.at[pl.ds(re_, REM)], ea16)

        _remap(idx_d16, REM, pbase, pend)

        @pl.when(cid == 0)
        def _():
            _compute(xj16, ea16, ov16, REM, 0)

        @pl.when(cid == 1)
        def _():
            _compute(xj16, ea16, ov16, REM, H)

        pltpu.sync_copy(ov16, acc.at[idx_d16], add=True)

    def _mk_fin(lo_hbm, hi_hbm, obase):
        def f(r0, rows):
            pltpu.sync_copy(acc.at[pl.ds(r0, rows)], fin.at[pl.ds(0, rows)])

            def _frow(r, c):
                for k in range(4):
                    w = fin[r, pl.ds(k * 16, 16)]
                    s = fin[r, pl.ds(H + k * 16, 16)]
                    res[r, pl.ds(k * 16, 16)] = w / (s + 1e-16)
                return c
            lax.fori_loop(0, rows, _frow, 0)

            ro = pl.multiple_of(obase + r0, 8)

            @pl.when(cid == 0)
            def _():
                pltpu.sync_copy(res.at[pl.ds(0, rows)],
                                lo_hbm.at[pl.ds(ro, rows)])

            @pl.when(cid == 1)
            def _():
                pltpu.sync_copy(res.at[pl.ds(0, rows)],
                                hi_hbm.at[pl.ds(ro, rows)])
        return f

    convs = (
        (0, e0lo, e0hi, a0lo, a0hi),
        (E2, e1lo, e1hi, a1lo, a1hi),
    )
    for link_off, elo, ehi, alo, ahi in convs:
        def _phase(ph, c):
            pbase = ph * P1
            pend = jnp.minimum(pbase + P1, N)
            _zero_ov()
            _stripe(ph, lambda r0, rows: pltpu.sync_copy(
                ov0.at[pl.ds(0, rows)], acc.at[pl.ds(r0, rows)]))
            plsc.subcore_barrier()
            _scan(link_off, elo, ehi, pbase, pend)
            plsc.subcore_barrier()
            _stripe(ph, _mk_fin(alo, ahi, pbase))
            plsc.subcore_barrier()
            return c
        lax.fori_loop(0, NPH, _phase, 0)


def _edge_call(ei1d, x, e0lo, e0hi, e1lo, e1hi):
    mesh = plsc.VectorSubcoreMesh(
        core_axis_name="c", subcore_axis_name="s",
        num_cores=NC, num_subcores=NS)
    return pl.kernel(
        _edge_body,
        name="edge_pass",
        compiler_params=pltpu.CompilerParams(use_tc_tiling_on_sc=False),
        out_type=[jax.ShapeDtypeStruct((N, H), jnp.float32)
                  for _ in range(4)],
        mesh=mesh,
        scratch_types=[
            pltpu.VMEM((EB,), jnp.int32),
            pltpu.VMEM((EB,), jnp.int32),
            pltpu.VMEM((EB,), jnp.int32),
            pltpu.VMEM((EB,), jnp.int32),
            pltpu.VMEM((EB, D), jnp.float32),
            pltpu.VMEM((EB, D), jnp.float32),
            pltpu.VMEM((EB, H), jnp.float32),
            pltpu.VMEM((EB, H), jnp.float32),
            pltpu.VMEM((EB, D), jnp.float32),
            pltpu.VMEM((EB, D), jnp.float32),
            pltpu.VMEM((REM,), jnp.int32),
            pltpu.VMEM((REM,), jnp.int32),
            pltpu.VMEM((REM, D), jnp.float32),
            pltpu.VMEM((REM, H), jnp.float32),
            pltpu.VMEM((REM, D), jnp.float32),
            pltpu.VMEM_SHARED((ACC_R, D), jnp.float32),
            pltpu.SemaphoreType.DMA,
            pltpu.SemaphoreType.DMA,
            pltpu.SemaphoreType.DMA,
        ],
    )(ei1d, x, e0lo, e0hi, e1lo, e1hi)


def _layer_body(last, x_ref, a0lo_ref, a0hi_ref, a1lo_ref, a1hi_ref,
                w1t_ref, g_ref, b_ref, w2t_ref, dep_ref, out_ref, accv):
    c = pl.program_id(0)
    is0 = (c == 0)
    alo = jnp.where(is0, a0lo_ref[...], a1lo_ref[...])
    ahi = jnp.where(is0, a0hi_ref[...], a1hi_ref[...])
    agg = jnp.concatenate([alo, ahi], axis=1)
    out = agg + x_ref[...]
    h = jnp.dot(out, w1t_ref[0], preferred_element_type=jnp.float32)
    m = jnp.mean(h, axis=0, keepdims=True)
    d = h - m
    v = jnp.mean(d * d, axis=0, keepdims=True)
    hn = d / jnp.sqrt(v + 1e-5) * g_ref[0] + b_ref[0]
    hr = jnp.maximum(hn, 0.0)
    y = jnp.dot(hr, w2t_ref[0], preferred_element_type=jnp.float32)

    @pl.when(c == 0)
    def _():
        accv[...] = y

    @pl.when(c == 1)
    def _():
        t = accv[...] + y
        if last:
            out_ref[...] = t + dep_ref[0, 0]
        else:
            out_ref[...] = jnp.where(t > 0, t, 0.01 * t)


def _layer_call(layer, last, x, a0lo, a0hi, a1lo, a1hi, W1T, g3, b3, W2T,
                dep):
    return pl.pallas_call(
        functools.partial(_layer_body, last),
        grid=(2,),
        in_specs=[
            pl.BlockSpec((N, D), lambda c: (0, 0)),
            pl.BlockSpec((N, H), lambda c: (0, 0)),
            pl.BlockSpec((N, H), lambda c: (0, 0)),
            pl.BlockSpec((N, H), lambda c: (0, 0)),
            pl.BlockSpec((N, H), lambda c: (0, 0)),
            pl.BlockSpec((1, D, 2 * D), lambda c: (2 * layer + c, 0, 0)),
            pl.BlockSpec((1, 1, 2 * D), lambda c: (2 * layer + c, 0, 0)),
            pl.BlockSpec((1, 1, 2 * D), lambda c: (2 * layer + c, 0, 0)),
            pl.BlockSpec((1, 2 * D, D), lambda c: (2 * layer + c, 0, 0)),
            pl.BlockSpec(memory_space=pltpu.SMEM),
        ],
        out_specs=pl.BlockSpec((N, D), lambda c: (0, 0)),
        out_shape=jax.ShapeDtypeStruct((N, D), jnp.float32),
        scratch_shapes=[pltpu.VMEM((N, D), jnp.float32)],
        compiler_params=pltpu.CompilerParams(
            dimension_semantics=("arbitrary",)),
    )(x, a0lo, a0hi, a1lo, a1hi, W1T, g3, b3, W2T, dep)


def kernel(x_hex, ei_flat, ea_flat, lengths, We, W1, gamma, beta, W2):
    ei1d = ei_flat.astype(jnp.int32).reshape(2 * DST_OFF)
    ea3 = ea_flat.reshape(1, 2 * E2, 16)

    WeT = jnp.transpose(We, (0, 2, 1))
    W1T = jnp.transpose(W1, (0, 2, 1))
    W2T = jnp.transpose(W2, (0, 2, 1))
    g3 = gamma.reshape(4, 1, 2 * D)
    b3 = beta.reshape(4, 1, 2 * D)
    dep = (lengths[0] + lengths[1] - 2 * E2).astype(jnp.float32).reshape(1, 1)

    eattrs = [_eattr_call(c, ea3, WeT) for c in range(4)]

    x = x_hex
    for layer in range(2):
        e0lo, e0hi = eattrs[2 * layer + 0]
        e1lo, e1hi = eattrs[2 * layer + 1]
        a0lo, a0hi, a1lo, a1hi = _edge_call(ei1d, x, e0lo, e0hi, e1lo, e1hi)
        x = _layer_call(layer, layer == 1, x, a0lo, a0hi, a1lo, a1hi,
                        W1T, g3, b3, W2T, dep)
    return x


# speedup vs baseline: 2.2021x; 2.2021x over previous
import functools

import jax
import jax.numpy as jnp
from jax import lax
from jax.experimental import pallas as pl
from jax.experimental.pallas import tpu as pltpu
from jax.experimental.pallas import tpu_sc as plsc

N = 10000
D = 128
H = 64
E2 = 160000
EB = 128
NC = 2
NS = 16
TILE_E = E2 // NS
NBLK = TILE_E // EB
REM = TILE_E % EB
DST_OFF = 2 * E2
ACC_R = 5120
P1 = 5104
DUMMY = P1
NPH = 2


def _eattr_body(ea_ref, wet_ref, lo_ref, hi_ref):
    r = jnp.dot(ea_ref[0], wet_ref[0], preferred_element_type=jnp.float32)
    lo_ref[...] = r[:, :H]
    hi_ref[...] = r[:, H:]


def _eattr_call(conv, ea3, WeT):
    Be = 8000
    nb = E2 // Be
    link_off = (conv % 2) * nb
    return pl.pallas_call(
        _eattr_body,
        grid=(nb,),
        in_specs=[
            pl.BlockSpec((1, Be, 16), lambda e: (0, link_off + e, 0)),
            pl.BlockSpec((1, 16, D), lambda e: (conv, 0, 0)),
        ],
        out_specs=[
            pl.BlockSpec((Be, H), lambda e: (e, 0)),
            pl.BlockSpec((Be, H), lambda e: (e, 0)),
        ],
        out_shape=[
            jax.ShapeDtypeStruct((E2, H), jnp.float32),
            jax.ShapeDtypeStruct((E2, H), jnp.float32),
        ],
        compiler_params=pltpu.CompilerParams(
            dimension_semantics=("arbitrary",)),
    )(ea3, WeT)


def _edge_body(ei_hbm, x_hbm, e0lo, e0hi, e1lo, e1hi,
               a0lo, a0hi, a1lo, a1hi,
               idx_s0, idx_s1, idx_d0, idx_d1, xj0, xj1, ea0, ea1, ov0, ov1,
               idx_s16, idx_d16, xj16, ea16, ov16,
               acc, g0, g1, ss):
    cid = lax.axis_index("c")
    sid = lax.axis_index("s")
    fin = xj0
    res = ea0

    def _stripe(nrows, fn):
        full = nrows // 128
        rem = nrows % 128
        for q in range((full + NS - 1) // NS):
            cix = q * NS + sid

            @pl.when(cix < full)
            def _():
                fn(pl.multiple_of(cix * 128, 128), 128)
        if rem:

            @pl.when(sid == full % NS)
            def _():
                fn(full * 128, rem)

    def _zero_ov():
        def _zrow(r, c):
            for k in range(8):
                ov0[r, pl.ds(k * 16, 16)] = jnp.zeros((16,), jnp.float32)
            return c
        lax.fori_loop(0, EB, _zrow, 0)

    def _compute(xjr, ear, ovr, nrows, xoff):
        def _row(r, c2):
            for k in range(4):
                xv = xjr[r, pl.ds(xoff + k * 16, 16)]
                ev = ear[r, pl.ds(k * 16, 16)]
                m = jnp.maximum(xv + ev, 0.0) + 1e-7
                ex = jnp.exp(m)
                ovr[r, pl.ds(H + k * 16, 16)] = ex
                ovr[r, pl.ds(k * 16, 16)] = m * ex
            return c2
        lax.fori_loop(0, nrows, _row, 0)

    def _remap(idxr, nrows, pbase, pend):
        def _rm(t, c2):
            dv = idxr[pl.ds(t * 16, 16)]
            ok = (dv >= pbase) & (dv < pend)
            idxr[pl.ds(t * 16, 16)] = jnp.where(
                ok, dv - pbase, jnp.full((16,), DUMMY, jnp.int32))
            return c2
        lax.fori_loop(0, nrows // 16, _rm, 0)

    def _scan(link_off, elo_hbm, ehi_hbm, pbase, pend):
        ebase = sid * TILE_E

        def _issue(blk, idx_s, xj, ea, gsem):
            bi = pl.multiple_of(link_off + ebase + blk * EB, 8)
            be = pl.multiple_of(ebase + blk * EB, 8)
            pltpu.sync_copy(ei_hbm.at[pl.ds(bi, EB)], idx_s)
            pltpu.async_copy(x_hbm.at[idx_s], xj, gsem)

            @pl.when(cid == 0)
            def _():
                pltpu.async_copy(elo_hbm.at[pl.ds(be, EB)], ea, gsem)

            @pl.when(cid == 1)
            def _():
                pltpu.async_copy(ehi_hbm.at[pl.ds(be, EB)], ea, gsem)

        def _wait_in(xj, ea, gsem):
            pltpu.make_async_copy(x_hbm.at[pl.ds(0, EB)], xj, gsem).wait()
            pltpu.make_async_copy(elo_hbm.at[pl.ds(0, EB)], ea, gsem).wait()

        def _wait_sc(ov):
            pltpu.make_async_copy(ov, acc.at[pl.ds(0, EB)], ss).wait()

        def _load_d(blk, idx_d):
            bi = pl.multiple_of(link_off + ebase + blk * EB, 8)
            pltpu.sync_copy(ei_hbm.at[pl.ds(DST_OFF + bi, EB)], idx_d)

        def _comp(xj, ea, ov):
            @pl.when(cid == 0)
            def _():
                _compute(xj, ea, ov, EB, 0)

            @pl.when(cid == 1)
            def _():
                _compute(xj, ea, ov, EB, H)

        def _blk(j, c):
            bi = pl.multiple_of(link_off + ebase + j * EB, 8)
            be = pl.multiple_of(ebase + j * EB, 8)
            pltpu.sync_copy(ei_hbm.at[pl.ds(bi, EB)], idx_s0)
            pltpu.sync_copy(ei_hbm.at[pl.ds(DST_OFF + bi, EB)], idx_d0)
            pltpu.async_copy(x_hbm.at[idx_s0], xj0, g0).wait()

            @pl.when(cid == 0)
            def _():
                pltpu.sync_copy(elo_hbm.at[pl.ds(be, EB)], ea0)

            @pl.when(cid == 1)
            def _():
                pltpu.sync_copy(ehi_hbm.at[pl.ds(be, EB)], ea0)

            _remap(idx_d0, EB, pbase, pend)
            _comp(xj0, ea0, ov0)
            pltpu.sync_copy(ov0, acc.at[idx_d0], add=True)
            return c
        lax.fori_loop(0, NBLK, _blk, 0)

        re_ = pl.multiple_of(ebase + NBLK * EB, 8)
        ri = pl.multiple_of(link_off + ebase + NBLK * EB, 8)
        pltpu.sync_copy(ei_hbm.at[pl.ds(ri, REM)], idx_s16)
        pltpu.sync_copy(ei_hbm.at[pl.ds(DST_OFF + ri, REM)], idx_d16)
        pltpu.async_copy(x_hbm.at[idx_s16], xj16, ss).wait()

        @pl.when(cid == 0)
        def _():
            pltpu.sync_copy(elo_hbm.at[pl.ds(re_, REM)], ea16)

        @pl.when(cid == 1)
        def _():
            pltpu.sync_copy(ehi_hbm.at[pl.ds(re_, REM)], ea16)

        _remap(idx_d16, REM, pbase, pend)

        @pl.when(cid == 0)
        def _():
            _compute(xj16, ea16, ov16, REM, 0)

        @pl.when(cid == 1)
        def _():
            _compute(xj16, ea16, ov16, REM, H)

        pltpu.sync_copy(ov16, acc.at[idx_d16], add=True)

    def _mk_fin(lo_hbm, hi_hbm, obase):
        def f(r0, rows):
            pltpu.sync_copy(acc.at[pl.ds(r0, rows)], fin.at[pl.ds(0, rows)])

            def _frow(r, c):
                for k in range(4):
                    w = fin[r, pl.ds(k * 16, 16)]
                    s = fin[r, pl.ds(H + k * 16, 16)]
                    res[r, pl.ds(k * 16, 16)] = w / (s + 1e-16)
                return c
            lax.fori_loop(0, rows, _frow, 0)

            ro = pl.multiple_of(obase + r0, 8)

            @pl.when(cid == 0)
            def _():
                pltpu.sync_copy(res.at[pl.ds(0, rows)],
                                lo_hbm.at[pl.ds(ro, rows)])

            @pl.when(cid == 1)
            def _():
                pltpu.sync_copy(res.at[pl.ds(0, rows)],
                                hi_hbm.at[pl.ds(ro, rows)])
        return f

    convs = (
        (0, e0lo, e0hi, a0lo, a0hi),
        (E2, e1lo, e1hi, a1lo, a1hi),
    )
    for link_off, elo, ehi, alo, ahi in convs:
        for pbase in (0, P1):
            pend = min(pbase + P1, N)
            nrows = pend - pbase
            _zero_ov()
            _stripe(nrows, lambda r0, rows: pltpu.sync_copy(
                ov0.at[pl.ds(0, rows)], acc.at[pl.ds(r0, rows)]))
            plsc.subcore_barrier()
            _scan(link_off, elo, ehi, pbase, pend)
            plsc.subcore_barrier()
            _stripe(nrows, _mk_fin(alo, ahi, pbase))
            plsc.subcore_barrier()


def _edge_call(ei1d, x, e0lo, e0hi, e1lo, e1hi):
    mesh = plsc.VectorSubcoreMesh(
        core_axis_name="c", subcore_axis_name="s",
        num_cores=NC, num_subcores=NS)
    return pl.kernel(
        _edge_body,
        name="edge_pass",
        out_type=[jax.ShapeDtypeStruct((N, H), jnp.float32)
                  for _ in range(4)],
        mesh=mesh,
        scratch_types=[
            pltpu.VMEM((EB,), jnp.int32),
            pltpu.VMEM((EB,), jnp.int32),
            pltpu.VMEM((EB,), jnp.int32),
            pltpu.VMEM((EB,), jnp.int32),
            pltpu.VMEM((EB, D), jnp.float32),
            pltpu.VMEM((EB, D), jnp.float32),
            pltpu.VMEM((EB, H), jnp.float32),
            pltpu.VMEM((EB, H), jnp.float32),
            pltpu.VMEM((EB, D), jnp.float32),
            pltpu.VMEM((EB, D), jnp.float32),
            pltpu.VMEM((REM,), jnp.int32),
            pltpu.VMEM((REM,), jnp.int32),
            pltpu.VMEM((REM, D), jnp.float32),
            pltpu.VMEM((REM, H), jnp.float32),
            pltpu.VMEM((REM, D), jnp.float32),
            pltpu.VMEM_SHARED((ACC_R, D), jnp.float32),
            pltpu.SemaphoreType.DMA,
            pltpu.SemaphoreType.DMA,
            pltpu.SemaphoreType.DMA,
        ],
    )(ei1d, x, e0lo, e0hi, e1lo, e1hi)


def _layer_body(last, x_ref, a0lo_ref, a0hi_ref, a1lo_ref, a1hi_ref,
                w1t_ref, g_ref, b_ref, w2t_ref, dep_ref, out_ref, accv):
    c = pl.program_id(0)
    is0 = (c == 0)
    alo = jnp.where(is0, a0lo_ref[...], a1lo_ref[...])
    ahi = jnp.where(is0, a0hi_ref[...], a1hi_ref[...])
    agg = jnp.concatenate([alo, ahi], axis=1)
    out = agg + x_ref[...]
    h = jnp.dot(out, w1t_ref[0], preferred_element_type=jnp.float32)
    m = jnp.mean(h, axis=0, keepdims=True)
    d = h - m
    v = jnp.mean(d * d, axis=0, keepdims=True)
    hn = d / jnp.sqrt(v + 1e-5) * g_ref[0] + b_ref[0]
    hr = jnp.maximum(hn, 0.0)
    y = jnp.dot(hr, w2t_ref[0], preferred_element_type=jnp.float32)

    @pl.when(c == 0)
    def _():
        accv[...] = y

    @pl.when(c == 1)
    def _():
        t = accv[...] + y
        if last:
            out_ref[...] = t + dep_ref[0, 0]
        else:
            out_ref[...] = jnp.where(t > 0, t, 0.01 * t)


def _layer_call(layer, last, x, a0lo, a0hi, a1lo, a1hi, W1T, g3, b3, W2T,
                dep):
    return pl.pallas_call(
        functools.partial(_layer_body, last),
        grid=(2,),
        in_specs=[
            pl.BlockSpec((N, D), lambda c: (0, 0)),
            pl.BlockSpec((N, H), lambda c: (0, 0)),
            pl.BlockSpec((N, H), lambda c: (0, 0)),
            pl.BlockSpec((N, H), lambda c: (0, 0)),
            pl.BlockSpec((N, H), lambda c: (0, 0)),
            pl.BlockSpec((1, D, 2 * D), lambda c: (2 * layer + c, 0, 0)),
            pl.BlockSpec((1, 1, 2 * D), lambda c: (2 * layer + c, 0, 0)),
            pl.BlockSpec((1, 1, 2 * D), lambda c: (2 * layer + c, 0, 0)),
            pl.BlockSpec((1, 2 * D, D), lambda c: (2 * layer + c, 0, 0)),
            pl.BlockSpec(memory_space=pltpu.SMEM),
        ],
        out_specs=pl.BlockSpec((N, D), lambda c: (0, 0)),
        out_shape=jax.ShapeDtypeStruct((N, D), jnp.float32),
        scratch_shapes=[pltpu.VMEM((N, D), jnp.float32)],
        compiler_params=pltpu.CompilerParams(
            dimension_semantics=("arbitrary",)),
    )(x, a0lo, a0hi, a1lo, a1hi, W1T, g3, b3, W2T, dep)


def kernel(x_hex, ei_flat, ea_flat, lengths, We, W1, gamma, beta, W2):
    ei1d = ei_flat.astype(jnp.int32).reshape(2 * DST_OFF)
    ea3 = ea_flat.reshape(1, 2 * E2, 16)

    WeT = jnp.transpose(We, (0, 2, 1))
    W1T = jnp.transpose(W1, (0, 2, 1))
    W2T = jnp.transpose(W2, (0, 2, 1))
    g3 = gamma.reshape(4, 1, 2 * D)
    b3 = beta.reshape(4, 1, 2 * D)
    dep = (lengths[0] + lengths[1] - 2 * E2).astype(jnp.float32).reshape(1, 1)

    eattrs = [_eattr_call(c, ea3, WeT) for c in range(4)]

    x = x_hex
    for layer in range(2):
        e0lo, e0hi = eattrs[2 * layer + 0]
        e1lo, e1hi = eattrs[2 * layer + 1]
        a0lo, a0hi, a1lo, a1hi = _edge_call(ei1d, x, e0lo, e0hi, e1lo, e1hi)
        x = _layer_call(layer, layer == 1, x, a0lo, a0hi, a1lo, a1hi,
                        W1T, g3, b3, W2T, dep)
    return x


# speedup vs baseline: 2.7559x; 1.2515x over previous
import functools

import jax
import jax.numpy as jnp
from jax import lax
from jax.experimental import pallas as pl
from jax.experimental.pallas import tpu as pltpu
from jax.experimental.pallas import tpu_sc as plsc

N = 10000
D = 128
H = 64
E2 = 160000
EB = 128
NC = 2
NS = 16
TILE_E = E2 // NS
NBLK = TILE_E // EB
REM = TILE_E % EB
DST_OFF = 2 * E2
ACC_R = 5120
P1 = 5104
DUMMY = P1
NPH = 2


def _eattr_body(ea_ref, wet_ref, lo_ref, hi_ref):
    r = jnp.dot(ea_ref[0], wet_ref[0], preferred_element_type=jnp.float32)
    lo_ref[...] = r[:, :H]
    hi_ref[...] = r[:, H:]


def _eattr_call(conv, ea3, WeT):
    Be = 8000
    nb = E2 // Be
    link_off = (conv % 2) * nb
    return pl.pallas_call(
        _eattr_body,
        grid=(nb,),
        in_specs=[
            pl.BlockSpec((1, Be, 16), lambda e: (0, link_off + e, 0)),
            pl.BlockSpec((1, 16, D), lambda e: (conv, 0, 0)),
        ],
        out_specs=[
            pl.BlockSpec((Be, H), lambda e: (e, 0)),
            pl.BlockSpec((Be, H), lambda e: (e, 0)),
        ],
        out_shape=[
            jax.ShapeDtypeStruct((E2, H), jnp.float32),
            jax.ShapeDtypeStruct((E2, H), jnp.float32),
        ],
        compiler_params=pltpu.CompilerParams(
            dimension_semantics=("arbitrary",)),
    )(ea3, WeT)


def _edge_body(ei_hbm, x_hbm, e0lo, e0hi, e1lo, e1hi,
               a0lo, a0hi, a1lo, a1hi,
               idx_s0, idx_s1, idx_d0, idx_d1, xj0, xj1, ea0, ea1, ov0, ov1,
               idx_s16, idx_d16, xj16, ea16, ov16,
               acc, g0, g1, ss):
    cid = lax.axis_index("c")
    sid = lax.axis_index("s")
    fin = xj0
    res = ea0

    def _stripe(nrows, fn):
        full = nrows // 128
        rem = nrows % 128
        for q in range((full + NS - 1) // NS):
            cix = q * NS + sid

            @pl.when(cix < full)
            def _():
                fn(pl.multiple_of(cix * 128, 128), 128)
        if rem:

            @pl.when(sid == full % NS)
            def _():
                fn(full * 128, rem)

    def _zero_ov():
        def _zrow(r, c):
            for k in range(8):
                ov0[r, pl.ds(k * 16, 16)] = jnp.zeros((16,), jnp.float32)
            return c
        lax.fori_loop(0, EB, _zrow, 0)

    def _compute(xjr, ear, ovr, nrows, xoff):
        def _row(r, c2):
            for k in range(4):
                xv = xjr[r, pl.ds(xoff + k * 16, 16)]
                ev = ear[r, pl.ds(k * 16, 16)]
                m = jnp.maximum(xv + ev, 0.0) + 1e-7
                ex = jnp.exp(m)
                ovr[r, pl.ds(H + k * 16, 16)] = ex
                ovr[r, pl.ds(k * 16, 16)] = m * ex
            return c2
        lax.fori_loop(0, nrows, _row, 0)

    def _remap(idxr, nrows, pbase, pend):
        def _rm(t, c2):
            dv = idxr[pl.ds(t * 16, 16)]
            ok = (dv >= pbase) & (dv < pend)
            idxr[pl.ds(t * 16, 16)] = jnp.where(
                ok, dv - pbase, jnp.full((16,), DUMMY, jnp.int32))
            return c2
        lax.fori_loop(0, nrows // 16, _rm, 0)

    def _scan(link_off, elo_hbm, ehi_hbm, pbase, pend):
        ebase = sid * TILE_E

        def _issue(blk, idx_s, xj, ea, gsem):
            bi = pl.multiple_of(link_off + ebase + blk * EB, 8)
            be = pl.multiple_of(ebase + blk * EB, 8)
            pltpu.sync_copy(ei_hbm.at[pl.ds(bi, EB)], idx_s)
            pltpu.async_copy(x_hbm.at[idx_s], xj, gsem)

            @pl.when(cid == 0)
            def _():
                pltpu.async_copy(elo_hbm.at[pl.ds(be, EB)], ea, gsem)

            @pl.when(cid == 1)
            def _():
                pltpu.async_copy(ehi_hbm.at[pl.ds(be, EB)], ea, gsem)

        def _wait_in(xj, ea, gsem):
            pltpu.make_async_copy(x_hbm.at[pl.ds(0, EB)], xj, gsem).wait()
            pltpu.make_async_copy(elo_hbm.at[pl.ds(0, EB)], ea, gsem).wait()

        def _wait_sc(ov):
            pltpu.make_async_copy(ov, acc.at[pl.ds(0, EB)], ss).wait()

        def _load_d(blk, idx_d):
            bi = pl.multiple_of(link_off + ebase + blk * EB, 8)
            pltpu.sync_copy(ei_hbm.at[pl.ds(DST_OFF + bi, EB)], idx_d)

        def _comp(xj, ea, ov):
            @pl.when(cid == 0)
            def _():
                _compute(xj, ea, ov, EB, 0)

            @pl.when(cid == 1)
            def _():
                _compute(xj, ea, ov, EB, H)

        def _blk(j, c):
            bi = pl.multiple_of(link_off + ebase + j * EB, 8)
            be = pl.multiple_of(ebase + j * EB, 8)
            pltpu.sync_copy(ei_hbm.at[pl.ds(bi, EB)], idx_s0)
            pltpu.async_copy(x_hbm.at[idx_s0], xj0, g0)

            @pl.when(cid == 0)
            def _():
                pltpu.async_copy(elo_hbm.at[pl.ds(be, EB)], ea0, g0)

            @pl.when(cid == 1)
            def _():
                pltpu.async_copy(ehi_hbm.at[pl.ds(be, EB)], ea0, g0)

            pltpu.sync_copy(ei_hbm.at[pl.ds(DST_OFF + bi, EB)], idx_d0)
            _remap(idx_d0, EB, pbase, pend)
            _wait_in(xj0, ea0, g0)
            _comp(xj0, ea0, ov0)
            pltpu.sync_copy(ov0, acc.at[idx_d0], add=True)
            return c
        lax.fori_loop(0, NBLK, _blk, 0)

        re_ = pl.multiple_of(ebase + NBLK * EB, 8)
        ri = pl.multiple_of(link_off + ebase + NBLK * EB, 8)
        pltpu.sync_copy(ei_hbm.at[pl.ds(ri, REM)], idx_s16)
        pltpu.sync_copy(ei_hbm.at[pl.ds(DST_OFF + ri, REM)], idx_d16)
        pltpu.async_copy(x_hbm.at[idx_s16], xj16, ss).wait()

        @pl.when(cid == 0)
        def _():
            pltpu.sync_copy(elo_hbm.at[pl.ds(re_, REM)], ea16)

        @pl.when(cid == 1)
        def _():
            pltpu.sync_copy(ehi_hbm.at[pl.ds(re_, REM)], ea16)

        _remap(idx_d16, REM, pbase, pend)

        @pl.when(cid == 0)
        def _():
            _compute(xj16, ea16, ov16, REM, 0)

        @pl.when(cid == 1)
        def _():
            _compute(xj16, ea16, ov16, REM, H)

        pltpu.sync_copy(ov16, acc.at[idx_d16], add=True)

    def _mk_fin(lo_hbm, hi_hbm, obase):
        def f(r0, rows):
            pltpu.sync_copy(acc.at[pl.ds(r0, rows)], fin.at[pl.ds(0, rows)])

            def _frow(r, c):
                for k in range(4):
                    w = fin[r, pl.ds(k * 16, 16)]
                    s = fin[r, pl.ds(H + k * 16, 16)]
                    res[r, pl.ds(k * 16, 16)] = w / (s + 1e-16)
                return c
            lax.fori_loop(0, rows, _frow, 0)

            ro = pl.multiple_of(obase + r0, 8)

            @pl.when(cid == 0)
            def _():
                pltpu.sync_copy(res.at[pl.ds(0, rows)],
                                lo_hbm.at[pl.ds(ro, rows)])

            @pl.when(cid == 1)
            def _():
                pltpu.sync_copy(res.at[pl.ds(0, rows)],
                                hi_hbm.at[pl.ds(ro, rows)])
        return f

    convs = (
        (0, e0lo, e0hi, a0lo, a0hi),
        (E2, e1lo, e1hi, a1lo, a1hi),
    )
    for link_off, elo, ehi, alo, ahi in convs:
        for pbase in (0, P1):
            pend = min(pbase + P1, N)
            nrows = pend - pbase
            _zero_ov()
            _stripe(nrows, lambda r0, rows: pltpu.sync_copy(
                ov0.at[pl.ds(0, rows)], acc.at[pl.ds(r0, rows)]))
            plsc.subcore_barrier()
            _scan(link_off, elo, ehi, pbase, pend)
            plsc.subcore_barrier()
            _stripe(nrows, _mk_fin(alo, ahi, pbase))
            plsc.subcore_barrier()


def _edge_call(ei1d, x, e0lo, e0hi, e1lo, e1hi):
    mesh = plsc.VectorSubcoreMesh(
        core_axis_name="c", subcore_axis_name="s",
        num_cores=NC, num_subcores=NS)
    return pl.kernel(
        _edge_body,
        name="edge_pass",
        out_type=[jax.ShapeDtypeStruct((N, H), jnp.float32)
                  for _ in range(4)],
        mesh=mesh,
        scratch_types=[
            pltpu.VMEM((EB,), jnp.int32),
            pltpu.VMEM((EB,), jnp.int32),
            pltpu.VMEM((EB,), jnp.int32),
            pltpu.VMEM((EB,), jnp.int32),
            pltpu.VMEM((EB, D), jnp.float32),
            pltpu.VMEM((EB, D), jnp.float32),
            pltpu.VMEM((EB, H), jnp.float32),
            pltpu.VMEM((EB, H), jnp.float32),
            pltpu.VMEM((EB, D), jnp.float32),
            pltpu.VMEM((EB, D), jnp.float32),
            pltpu.VMEM((REM,), jnp.int32),
            pltpu.VMEM((REM,), jnp.int32),
            pltpu.VMEM((REM, D), jnp.float32),
            pltpu.VMEM((REM, H), jnp.float32),
            pltpu.VMEM((REM, D), jnp.float32),
            pltpu.VMEM_SHARED((ACC_R, D), jnp.float32),
            pltpu.SemaphoreType.DMA,
            pltpu.SemaphoreType.DMA,
            pltpu.SemaphoreType.DMA,
        ],
    )(ei1d, x, e0lo, e0hi, e1lo, e1hi)


def _layer_body(last, x_ref, a0lo_ref, a0hi_ref, a1lo_ref, a1hi_ref,
                w1t_ref, g_ref, b_ref, w2t_ref, dep_ref, out_ref, accv):
    c = pl.program_id(0)
    is0 = (c == 0)
    alo = jnp.where(is0, a0lo_ref[...], a1lo_ref[...])
    ahi = jnp.where(is0, a0hi_ref[...], a1hi_ref[...])
    agg = jnp.concatenate([alo, ahi], axis=1)
    out = agg + x_ref[...]
    h = jnp.dot(out, w1t_ref[0], preferred_element_type=jnp.float32)
    m = jnp.mean(h, axis=0, keepdims=True)
    d = h - m
    v = jnp.mean(d * d, axis=0, keepdims=True)
    hn = d / jnp.sqrt(v + 1e-5) * g_ref[0] + b_ref[0]
    hr = jnp.maximum(hn, 0.0)
    y = jnp.dot(hr, w2t_ref[0], preferred_element_type=jnp.float32)

    @pl.when(c == 0)
    def _():
        accv[...] = y

    @pl.when(c == 1)
    def _():
        t = accv[...] + y
        if last:
            out_ref[...] = t + dep_ref[0, 0]
        else:
            out_ref[...] = jnp.where(t > 0, t, 0.01 * t)


def _layer_call(layer, last, x, a0lo, a0hi, a1lo, a1hi, W1T, g3, b3, W2T,
                dep):
    return pl.pallas_call(
        functools.partial(_layer_body, last),
        grid=(2,),
        in_specs=[
            pl.BlockSpec((N, D), lambda c: (0, 0)),
            pl.BlockSpec((N, H), lambda c: (0, 0)),
            pl.BlockSpec((N, H), lambda c: (0, 0)),
            pl.BlockSpec((N, H), lambda c: (0, 0)),
            pl.BlockSpec((N, H), lambda c: (0, 0)),
            pl.BlockSpec((1, D, 2 * D), lambda c: (2 * layer + c, 0, 0)),
            pl.BlockSpec((1, 1, 2 * D), lambda c: (2 * layer + c, 0, 0)),
            pl.BlockSpec((1, 1, 2 * D), lambda c: (2 * layer + c, 0, 0)),
            pl.BlockSpec((1, 2 * D, D), lambda c: (2 * layer + c, 0, 0)),
            pl.BlockSpec(memory_space=pltpu.SMEM),
        ],
        out_specs=pl.BlockSpec((N, D), lambda c: (0, 0)),
        out_shape=jax.ShapeDtypeStruct((N, D), jnp.float32),
        scratch_shapes=[pltpu.VMEM((N, D), jnp.float32)],
        compiler_params=pltpu.CompilerParams(
            dimension_semantics=("arbitrary",)),
    )(x, a0lo, a0hi, a1lo, a1hi, W1T, g3, b3, W2T, dep)


def kernel(x_hex, ei_flat, ea_flat, lengths, We, W1, gamma, beta, W2):
    ei1d = ei_flat.astype(jnp.int32).reshape(2 * DST_OFF)
    ea3 = ea_flat.reshape(1, 2 * E2, 16)

    WeT = jnp.transpose(We, (0, 2, 1))
    W1T = jnp.transpose(W1, (0, 2, 1))
    W2T = jnp.transpose(W2, (0, 2, 1))
    g3 = gamma.reshape(4, 1, 2 * D)
    b3 = beta.reshape(4, 1, 2 * D)
    dep = (lengths[0] + lengths[1] - 2 * E2).astype(jnp.float32).reshape(1, 1)

    eattrs = [_eattr_call(c, ea3, WeT) for c in range(4)]

    x = x_hex
    for layer in range(2):
        e0lo, e0hi = eattrs[2 * layer + 0]
        e1lo, e1hi = eattrs[2 * layer + 1]
        a0lo, a0hi, a1lo, a1hi = _edge_call(ei1d, x, e0lo, e0hi, e1lo, e1hi)
        x = _layer_call(layer, layer == 1, x, a0lo, a0hi, a1lo, a1hi,
                        W1T, g3, b3, W2T, dep)
    return x


# speedup vs baseline: 3.1722x; 1.1510x over previous
import functools

import jax
import jax.numpy as jnp
from jax import lax
from jax.experimental import pallas as pl
from jax.experimental.pallas import tpu as pltpu
from jax.experimental.pallas import tpu_sc as plsc

N = 10000
D = 128
H = 64
E2 = 160000
EB = 128
NC = 2
NS = 16
TILE_E = E2 // NS
NBLK = TILE_E // EB
REM = TILE_E % EB
DST_OFF = 2 * E2
ACC_R = 5120
P1 = 5104
DUMMY = P1
NPH = 2


def _eattr_body(ea_ref, wet_ref, lo_ref, hi_ref):
    r = jnp.dot(ea_ref[0], wet_ref[0], preferred_element_type=jnp.float32)
    lo_ref[...] = r[:, :H]
    hi_ref[...] = r[:, H:]


def _eattr_call(conv, ea3, WeT):
    Be = 8000
    nb = E2 // Be
    link_off = (conv % 2) * nb
    return pl.pallas_call(
        _eattr_body,
        grid=(nb,),
        in_specs=[
            pl.BlockSpec((1, Be, 16), lambda e: (0, link_off + e, 0)),
            pl.BlockSpec((1, 16, D), lambda e: (conv, 0, 0)),
        ],
        out_specs=[
            pl.BlockSpec((Be, H), lambda e: (e, 0)),
            pl.BlockSpec((Be, H), lambda e: (e, 0)),
        ],
        out_shape=[
            jax.ShapeDtypeStruct((E2, H), jnp.float32),
            jax.ShapeDtypeStruct((E2, H), jnp.float32),
        ],
        compiler_params=pltpu.CompilerParams(
            dimension_semantics=("arbitrary",)),
    )(ea3, WeT)


def _edge_body(ei_hbm, x_hbm, e0lo, e0hi, e1lo, e1hi,
               a0lo, a0hi, a1lo, a1hi,
               idx_s0, idx_s1, idx_d0, idx_d1, xj0, xj1, ea0, ea1, ov0, ov1,
               idx_s16, idx_d16, xj16, ea16, ov16,
               acc, g0, g1, ss):
    cid = lax.axis_index("c")
    sid = lax.axis_index("s")
    fin = xj0
    res = ea0

    def _stripe(nrows, fn):
        full = nrows // 128
        rem = nrows % 128
        for q in range((full + NS - 1) // NS):
            cix = q * NS + sid

            @pl.when(cix < full)
            def _():
                fn(pl.multiple_of(cix * 128, 128), 128)
        if rem:

            @pl.when(sid == full % NS)
            def _():
                fn(full * 128, rem)

    def _zero_ov():
        def _zrow(r, c):
            for k in range(8):
                ov0[r, pl.ds(k * 16, 16)] = jnp.zeros((16,), jnp.float32)
            return c
        lax.fori_loop(0, EB, _zrow, 0)

    def _compute(xjr, ear, ovr, nrows, xoff):
        def _row(r, c2):
            for k in range(4):
                xv = xjr[r, pl.ds(xoff + k * 16, 16)]
                ev = ear[r, pl.ds(k * 16, 16)]
                m = jnp.maximum(xv + ev, 0.0) + 1e-7
                ex = jnp.exp(m)
                ovr[r, pl.ds(H + k * 16, 16)] = ex
                ovr[r, pl.ds(k * 16, 16)] = m * ex
            return c2
        lax.fori_loop(0, nrows, _row, 0)

    def _remap(idxr, nrows, pbase, pend):
        def _rm(t, c2):
            dv = idxr[pl.ds(t * 16, 16)]
            ok = (dv >= pbase) & (dv < pend)
            idxr[pl.ds(t * 16, 16)] = jnp.where(
                ok, dv - pbase, jnp.full((16,), DUMMY, jnp.int32))
            return c2
        lax.fori_loop(0, nrows // 16, _rm, 0)

    def _scan(link_off, elo_hbm, ehi_hbm, pbase, pend):
        ebase = sid * TILE_E

        def _issue(blk, idx_s, xj, ea, gsem):
            bi = pl.multiple_of(link_off + ebase + blk * EB, 8)
            be = pl.multiple_of(ebase + blk * EB, 8)
            pltpu.sync_copy(ei_hbm.at[pl.ds(bi, EB)], idx_s)
            pltpu.async_copy(x_hbm.at[idx_s], xj, gsem)

            @pl.when(cid == 0)
            def _():
                pltpu.async_copy(elo_hbm.at[pl.ds(be, EB)], ea, gsem)

            @pl.when(cid == 1)
            def _():
                pltpu.async_copy(ehi_hbm.at[pl.ds(be, EB)], ea, gsem)

        def _wait_in(xj, ea, gsem):
            pltpu.make_async_copy(x_hbm.at[pl.ds(0, EB)], xj, gsem).wait()
            pltpu.make_async_copy(elo_hbm.at[pl.ds(0, EB)], ea, gsem).wait()

        def _wait_sc(ov):
            pltpu.make_async_copy(ov, acc.at[pl.ds(0, EB)], ss).wait()

        def _load_d(blk, idx_d):
            bi = pl.multiple_of(link_off + ebase + blk * EB, 8)
            pltpu.sync_copy(ei_hbm.at[pl.ds(DST_OFF + bi, EB)], idx_d)

        def _comp(xj, ea, ov):
            @pl.when(cid == 0)
            def _():
                _compute(xj, ea, ov, EB, 0)

            @pl.when(cid == 1)
            def _():
                _compute(xj, ea, ov, EB, H)

        def _blk(j, c):
            bi = pl.multiple_of(link_off + ebase + j * EB, 8)
            be = pl.multiple_of(ebase + j * EB, 8)
            pltpu.sync_copy(ei_hbm.at[pl.ds(bi, EB)], idx_s0)
            pltpu.async_copy(x_hbm.at[idx_s0], xj0, g0)

            @pl.when(cid == 0)
            def _():
                pltpu.async_copy(elo_hbm.at[pl.ds(be, EB)], ea0, g0)

            @pl.when(cid == 1)
            def _():
                pltpu.async_copy(ehi_hbm.at[pl.ds(be, EB)], ea0, g0)

            @pl.when(j > 0)
            def _():
                _wait_sc(ov0)

            pltpu.sync_copy(ei_hbm.at[pl.ds(DST_OFF + bi, EB)], idx_d0)
            _remap(idx_d0, EB, pbase, pend)
            _wait_in(xj0, ea0, g0)
            _comp(xj0, ea0, ov0)
            pltpu.async_copy(ov0, acc.at[idx_d0], ss, add=True)
            return c
        lax.fori_loop(0, NBLK, _blk, 0)
        _wait_sc(ov0)

        re_ = pl.multiple_of(ebase + NBLK * EB, 8)
        ri = pl.multiple_of(link_off + ebase + NBLK * EB, 8)
        pltpu.sync_copy(ei_hbm.at[pl.ds(ri, REM)], idx_s16)
        pltpu.sync_copy(ei_hbm.at[pl.ds(DST_OFF + ri, REM)], idx_d16)
        pltpu.async_copy(x_hbm.at[idx_s16], xj16, ss).wait()

        @pl.when(cid == 0)
        def _():
            pltpu.sync_copy(elo_hbm.at[pl.ds(re_, REM)], ea16)

        @pl.when(cid == 1)
        def _():
            pltpu.sync_copy(ehi_hbm.at[pl.ds(re_, REM)], ea16)

        _remap(idx_d16, REM, pbase, pend)

        @pl.when(cid == 0)
        def _():
            _compute(xj16, ea16, ov16, REM, 0)

        @pl.when(cid == 1)
        def _():
            _compute(xj16, ea16, ov16, REM, H)

        pltpu.sync_copy(ov16, acc.at[idx_d16], add=True)

    def _mk_fin(lo_hbm, hi_hbm, obase):
        def f(r0, rows):
            pltpu.sync_copy(acc.at[pl.ds(r0, rows)], fin.at[pl.ds(0, rows)])

            def _frow(r, c):
                for k in range(4):
                    w = fin[r, pl.ds(k * 16, 16)]
                    s = fin[r, pl.ds(H + k * 16, 16)]
                    res[r, pl.ds(k * 16, 16)] = w / (s + 1e-16)
                return c
            lax.fori_loop(0, rows, _frow, 0)

            ro = pl.multiple_of(obase + r0, 8)

            @pl.when(cid == 0)
            def _():
                pltpu.sync_copy(res.at[pl.ds(0, rows)],
                                lo_hbm.at[pl.ds(ro, rows)])

            @pl.when(cid == 1)
            def _():
                pltpu.sync_copy(res.at[pl.ds(0, rows)],
                                hi_hbm.at[pl.ds(ro, rows)])
        return f

    convs = (
        (0, e0lo, e0hi, a0lo, a0hi),
        (E2, e1lo, e1hi, a1lo, a1hi),
    )
    for link_off, elo, ehi, alo, ahi in convs:
        for pbase in (0, P1):
            pend = min(pbase + P1, N)
            nrows = pend - pbase
            _zero_ov()
            _stripe(nrows, lambda r0, rows: pltpu.sync_copy(
                ov0.at[pl.ds(0, rows)], acc.at[pl.ds(r0, rows)]))
            plsc.subcore_barrier()
            _scan(link_off, elo, ehi, pbase, pend)
            plsc.subcore_barrier()
            _stripe(nrows, _mk_fin(alo, ahi, pbase))
            plsc.subcore_barrier()


def _edge_call(ei1d, x, e0lo, e0hi, e1lo, e1hi):
    mesh = plsc.VectorSubcoreMesh(
        core_axis_name="c", subcore_axis_name="s",
        num_cores=NC, num_subcores=NS)
    return pl.kernel(
        _edge_body,
        name="edge_pass",
        out_type=[jax.ShapeDtypeStruct((N, H), jnp.float32)
                  for _ in range(4)],
        mesh=mesh,
        scratch_types=[
            pltpu.VMEM((EB,), jnp.int32),
            pltpu.VMEM((EB,), jnp.int32),
            pltpu.VMEM((EB,), jnp.int32),
            pltpu.VMEM((EB,), jnp.int32),
            pltpu.VMEM((EB, D), jnp.float32),
            pltpu.VMEM((EB, D), jnp.float32),
            pltpu.VMEM((EB, H), jnp.float32),
            pltpu.VMEM((EB, H), jnp.float32),
            pltpu.VMEM((EB, D), jnp.float32),
            pltpu.VMEM((EB, D), jnp.float32),
            pltpu.VMEM((REM,), jnp.int32),
            pltpu.VMEM((REM,), jnp.int32),
            pltpu.VMEM((REM, D), jnp.float32),
            pltpu.VMEM((REM, H), jnp.float32),
            pltpu.VMEM((REM, D), jnp.float32),
            pltpu.VMEM_SHARED((ACC_R, D), jnp.float32),
            pltpu.SemaphoreType.DMA,
            pltpu.SemaphoreType.DMA,
            pltpu.SemaphoreType.DMA,
        ],
    )(ei1d, x, e0lo, e0hi, e1lo, e1hi)


def _layer_body(last, x_ref, a0lo_ref, a0hi_ref, a1lo_ref, a1hi_ref,
                w1t_ref, g_ref, b_ref, w2t_ref, dep_ref, out_ref, accv):
    c = pl.program_id(0)
    is0 = (c == 0)
    alo = jnp.where(is0, a0lo_ref[...], a1lo_ref[...])
    ahi = jnp.where(is0, a0hi_ref[...], a1hi_ref[...])
    agg = jnp.concatenate([alo, ahi], axis=1)
    out = agg + x_ref[...]
    h = jnp.dot(out, w1t_ref[0], preferred_element_type=jnp.float32)
    m = jnp.mean(h, axis=0, keepdims=True)
    d = h - m
    v = jnp.mean(d * d, axis=0, keepdims=True)
    hn = d / jnp.sqrt(v + 1e-5) * g_ref[0] + b_ref[0]
    hr = jnp.maximum(hn, 0.0)
    y = jnp.dot(hr, w2t_ref[0], preferred_element_type=jnp.float32)

    @pl.when(c == 0)
    def _():
        accv[...] = y

    @pl.when(c == 1)
    def _():
        t = accv[...] + y
        if last:
            out_ref[...] = t + dep_ref[0, 0]
        else:
            out_ref[...] = jnp.where(t > 0, t, 0.01 * t)


def _layer_call(layer, last, x, a0lo, a0hi, a1lo, a1hi, W1T, g3, b3, W2T,
                dep):
    return pl.pallas_call(
        functools.partial(_layer_body, last),
        grid=(2,),
        in_specs=[
            pl.BlockSpec((N, D), lambda c: (0, 0)),
            pl.BlockSpec((N, H), lambda c: (0, 0)),
            pl.BlockSpec((N, H), lambda c: (0, 0)),
            pl.BlockSpec((N, H), lambda c: (0, 0)),
            pl.BlockSpec((N, H), lambda c: (0, 0)),
            pl.BlockSpec((1, D, 2 * D), lambda c: (2 * layer + c, 0, 0)),
            pl.BlockSpec((1, 1, 2 * D), lambda c: (2 * layer + c, 0, 0)),
            pl.BlockSpec((1, 1, 2 * D), lambda c: (2 * layer + c, 0, 0)),
            pl.BlockSpec((1, 2 * D, D), lambda c: (2 * layer + c, 0, 0)),
            pl.BlockSpec(memory_space=pltpu.SMEM),
        ],
        out_specs=pl.BlockSpec((N, D), lambda c: (0, 0)),
        out_shape=jax.ShapeDtypeStruct((N, D), jnp.float32),
        scratch_shapes=[pltpu.VMEM((N, D), jnp.float32)],
        compiler_params=pltpu.CompilerParams(
            dimension_semantics=("arbitrary",)),
    )(x, a0lo, a0hi, a1lo, a1hi, W1T, g3, b3, W2T, dep)


def kernel(x_hex, ei_flat, ea_flat, lengths, We, W1, gamma, beta, W2):
    ei1d = ei_flat.astype(jnp.int32).reshape(2 * DST_OFF)
    ea3 = ea_flat.reshape(1, 2 * E2, 16)

    WeT = jnp.transpose(We, (0, 2, 1))
    W1T = jnp.transpose(W1, (0, 2, 1))
    W2T = jnp.transpose(W2, (0, 2, 1))
    g3 = gamma.reshape(4, 1, 2 * D)
    b3 = beta.reshape(4, 1, 2 * D)
    dep = (lengths[0] + lengths[1] - 2 * E2).astype(jnp.float32).reshape(1, 1)

    eattrs = [_eattr_call(c, ea3, WeT) for c in range(4)]

    x = x_hex
    for layer in range(2):
        e0lo, e0hi = eattrs[2 * layer + 0]
        e1lo, e1hi = eattrs[2 * layer + 1]
        a0lo, a0hi, a1lo, a1hi = _edge_call(ei1d, x, e0lo, e0hi, e1lo, e1hi)
        x = _layer_call(layer, layer == 1, x, a0lo, a0hi, a1lo, a1hi,
                        W1T, g3, b3, W2T, dep)
    return x


# speedup vs baseline: 3.3864x; 1.0675x over previous
import functools

import jax
import jax.numpy as jnp
from jax import lax
from jax.experimental import pallas as pl
from jax.experimental.pallas import tpu as pltpu
from jax.experimental.pallas import tpu_sc as plsc

N = 10000
D = 128
H = 64
E2 = 160000
EB = 128
NC = 2
NS = 16
TILE_E = E2 // NS
NBLK = TILE_E // EB
REM = TILE_E % EB
DST_OFF = 2 * E2
ACC_R = 5120
P1 = 5104
DUMMY = P1
NPH = 2


def _eattr_body(ea_ref, wet_ref, lo_ref, hi_ref):
    r = jnp.dot(ea_ref[0], wet_ref[0], preferred_element_type=jnp.float32)
    lo_ref[...] = r[:, :H]
    hi_ref[...] = r[:, H:]


def _eattr_call(conv, ea3, WeT):
    Be = 8000
    nb = E2 // Be
    link_off = (conv % 2) * nb
    return pl.pallas_call(
        _eattr_body,
        grid=(nb,),
        in_specs=[
            pl.BlockSpec((1, Be, 16), lambda e: (0, link_off + e, 0)),
            pl.BlockSpec((1, 16, D), lambda e: (conv, 0, 0)),
        ],
        out_specs=[
            pl.BlockSpec((Be, H), lambda e: (e, 0)),
            pl.BlockSpec((Be, H), lambda e: (e, 0)),
        ],
        out_shape=[
            jax.ShapeDtypeStruct((E2, H), jnp.float32),
            jax.ShapeDtypeStruct((E2, H), jnp.float32),
        ],
        compiler_params=pltpu.CompilerParams(
            dimension_semantics=("arbitrary",)),
    )(ea3, WeT)


def _edge_body(ei_hbm, x_hbm, e0lo, e0hi, e1lo, e1hi,
               a0lo, a0hi, a1lo, a1hi,
               idx_s0, idx_s1, idx_d0, idx_d1, xj0, xj1, ea0, ea1, ov0, ov1,
               idx_s16, idx_d16, xj16, ea16, ov16,
               acc, g0, g1, ss):
    cid = lax.axis_index("c")
    sid = lax.axis_index("s")
    fin = xj0
    res = ea0

    def _stripe(nrows, fn):
        full = nrows // 128
        rem = nrows % 128
        for q in range((full + NS - 1) // NS):
            cix = q * NS + sid

            @pl.when(cix < full)
            def _():
                fn(pl.multiple_of(cix * 128, 128), 128)
        if rem:

            @pl.when(sid == full % NS)
            def _():
                fn(full * 128, rem)

    def _zero_ov():
        def _zrow(r, c):
            for k in range(8):
                ov0[r, pl.ds(k * 16, 16)] = jnp.zeros((16,), jnp.float32)
            return c
        lax.fori_loop(0, EB, _zrow, 0)

    def _compute(xjr, ear, ovr, nrows, xoff, unroll=1):
        def _one(r):
            for k in range(4):
                xv = xjr[r, pl.ds(xoff + k * 16, 16)]
                ev = ear[r, pl.ds(k * 16, 16)]
                m = jnp.maximum(xv + ev, 0.0) + 1e-7
                ex = jnp.exp(m)
                ovr[r, pl.ds(H + k * 16, 16)] = ex
                ovr[r, pl.ds(k * 16, 16)] = m * ex

        def _row(r, c2):
            for u in range(unroll):
                _one(unroll * r + u)
            return c2
        lax.fori_loop(0, nrows // unroll, _row, 0)

    def _remap(idxr, nrows, pbase, pend):
        def _rm(t, c2):
            dv = idxr[pl.ds(t * 16, 16)]
            ok = (dv >= pbase) & (dv < pend)
            idxr[pl.ds(t * 16, 16)] = jnp.where(
                ok, dv - pbase, jnp.full((16,), DUMMY, jnp.int32))
            return c2
        lax.fori_loop(0, nrows // 16, _rm, 0)

    def _scan(link_off, elo_hbm, ehi_hbm, pbase, pend):
        ebase = sid * TILE_E

        def _issue(blk, idx_s, xj, ea, gsem):
            bi = pl.multiple_of(link_off + ebase + blk * EB, 8)
            be = pl.multiple_of(ebase + blk * EB, 8)
            pltpu.sync_copy(ei_hbm.at[pl.ds(bi, EB)], idx_s)
            pltpu.async_copy(x_hbm.at[idx_s], xj, gsem)

            @pl.when(cid == 0)
            def _():
                pltpu.async_copy(elo_hbm.at[pl.ds(be, EB)], ea, gsem)

            @pl.when(cid == 1)
            def _():
                pltpu.async_copy(ehi_hbm.at[pl.ds(be, EB)], ea, gsem)

        def _wait_in(xj, ea, gsem):
            pltpu.make_async_copy(x_hbm.at[pl.ds(0, EB)], xj, gsem).wait()
            pltpu.make_async_copy(elo_hbm.at[pl.ds(0, EB)], ea, gsem).wait()

        def _wait_sc(ov):
            pltpu.make_async_copy(ov, acc.at[pl.ds(0, EB)], ss).wait()

        def _load_d(blk, idx_d):
            bi = pl.multiple_of(link_off + ebase + blk * EB, 8)
            pltpu.sync_copy(ei_hbm.at[pl.ds(DST_OFF + bi, EB)], idx_d)

        def _comp(xj, ea, ov):
            @pl.when(cid == 0)
            def _():
                _compute(xj, ea, ov, EB, 0, unroll=2)

            @pl.when(cid == 1)
            def _():
                _compute(xj, ea, ov, EB, H, unroll=2)

        def _blk(j, c):
            bi = pl.multiple_of(link_off + ebase + j * EB, 8)
            be = pl.multiple_of(ebase + j * EB, 8)

            @pl.when(j > 0)
            def _():
                pltpu.make_async_copy(
                    ei_hbm.at[pl.ds(0, EB)], idx_s0, g1).wait()

            pltpu.async_copy(x_hbm.at[idx_s0], xj0, g0)

            @pl.when(cid == 0)
            def _():
                pltpu.async_copy(elo_hbm.at[pl.ds(be, EB)], ea0, g0)

            @pl.when(cid == 1)
            def _():
                pltpu.async_copy(ehi_hbm.at[pl.ds(be, EB)], ea0, g0)

            @pl.when(j > 0)
            def _():
                _wait_sc(ov0)

            pltpu.sync_copy(ei_hbm.at[pl.ds(DST_OFF + bi, EB)], idx_d0)
            _remap(idx_d0, EB, pbase, pend)
            _wait_in(xj0, ea0, g0)

            @pl.when(j < NBLK - 1)
            def _():
                bn = pl.multiple_of(link_off + ebase + (j + 1) * EB, 8)
                pltpu.async_copy(ei_hbm.at[pl.ds(bn, EB)], idx_s0, g1)

            _comp(xj0, ea0, ov0)
            pltpu.async_copy(ov0, acc.at[idx_d0], ss, add=True)
            return c
        pltpu.sync_copy(ei_hbm.at[pl.ds(pl.multiple_of(
            link_off + ebase, 8), EB)], idx_s0)
        lax.fori_loop(0, NBLK, _blk, 0)
        _wait_sc(ov0)

        re_ = pl.multiple_of(ebase + NBLK * EB, 8)
        ri = pl.multiple_of(link_off + ebase + NBLK * EB, 8)
        pltpu.sync_copy(ei_hbm.at[pl.ds(ri, REM)], idx_s16)
        pltpu.sync_copy(ei_hbm.at[pl.ds(DST_OFF + ri, REM)], idx_d16)
        pltpu.async_copy(x_hbm.at[idx_s16], xj16, ss).wait()

        @pl.when(cid == 0)
        def _():
            pltpu.sync_copy(elo_hbm.at[pl.ds(re_, REM)], ea16)

        @pl.when(cid == 1)
        def _():
            pltpu.sync_copy(ehi_hbm.at[pl.ds(re_, REM)], ea16)

        _remap(idx_d16, REM, pbase, pend)

        @pl.when(cid == 0)
        def _():
            _compute(xj16, ea16, ov16, REM, 0)

        @pl.when(cid == 1)
        def _():
            _compute(xj16, ea16, ov16, REM, H)

        pltpu.sync_copy(ov16, acc.at[idx_d16], add=True)

    def _mk_fin(lo_hbm, hi_hbm, obase):
        def f(r0, rows):
            pltpu.sync_copy(acc.at[pl.ds(r0, rows)], fin.at[pl.ds(0, rows)])

            def _frow(r, c):
                for k in range(4):
                    w = fin[r, pl.ds(k * 16, 16)]
                    s = fin[r, pl.ds(H + k * 16, 16)]
                    res[r, pl.ds(k * 16, 16)] = w / (s + 1e-16)
                return c
            lax.fori_loop(0, rows, _frow, 0)

            ro = pl.multiple_of(obase + r0, 8)

            @pl.when(cid == 0)
            def _():
                pltpu.sync_copy(res.at[pl.ds(0, rows)],
                                lo_hbm.at[pl.ds(ro, rows)])

            @pl.when(cid == 1)
            def _():
                pltpu.sync_copy(res.at[pl.ds(0, rows)],
                                hi_hbm.at[pl.ds(ro, rows)])
        return f

    convs = (
        (0, e0lo, e0hi, a0lo, a0hi),
        (E2, e1lo, e1hi, a1lo, a1hi),
    )
    for link_off, elo, ehi, alo, ahi in convs:
        for pbase in (0, P1):
            pend = min(pbase + P1, N)
            nrows = pend - pbase
            _zero_ov()
            _stripe(nrows, lambda r0, rows: pltpu.sync_copy(
                ov0.at[pl.ds(0, rows)], acc.at[pl.ds(r0, rows)]))
            plsc.subcore_barrier()
            _scan(link_off, elo, ehi, pbase, pend)
            plsc.subcore_barrier()
            _stripe(nrows, _mk_fin(alo, ahi, pbase))
            plsc.subcore_barrier()


def _edge_call(ei1d, x, e0lo, e0hi, e1lo, e1hi):
    mesh = plsc.VectorSubcoreMesh(
        core_axis_name="c", subcore_axis_name="s",
        num_cores=NC, num_subcores=NS)
    return pl.kernel(
        _edge_body,
        name="edge_pass",
        out_type=[jax.ShapeDtypeStruct((N, H), jnp.float32)
                  for _ in range(4)],
        mesh=mesh,
        scratch_types=[
            pltpu.VMEM((EB,), jnp.int32),
            pltpu.VMEM((EB,), jnp.int32),
            pltpu.VMEM((EB,), jnp.int32),
            pltpu.VMEM((EB,), jnp.int32),
            pltpu.VMEM((EB, D), jnp.float32),
            pltpu.VMEM((EB, D), jnp.float32),
            pltpu.VMEM((EB, H), jnp.float32),
            pltpu.VMEM((EB, H), jnp.float32),
            pltpu.VMEM((EB, D), jnp.float32),
            pltpu.VMEM((EB, D), jnp.float32),
            pltpu.VMEM((REM,), jnp.int32),
            pltpu.VMEM((REM,), jnp.int32),
            pltpu.VMEM((REM, D), jnp.float32),
            pltpu.VMEM((REM, H), jnp.float32),
            pltpu.VMEM((REM, D), jnp.float32),
            pltpu.VMEM_SHARED((ACC_R, D), jnp.float32),
            pltpu.SemaphoreType.DMA,
            pltpu.SemaphoreType.DMA,
            pltpu.SemaphoreType.DMA,
        ],
    )(ei1d, x, e0lo, e0hi, e1lo, e1hi)


def _layer_body(last, x_ref, a0lo_ref, a0hi_ref, a1lo_ref, a1hi_ref,
                w1t_ref, g_ref, b_ref, w2t_ref, dep_ref, out_ref, accv):
    c = pl.program_id(0)
    is0 = (c == 0)
    alo = jnp.where(is0, a0lo_ref[...], a1lo_ref[...])
    ahi = jnp.where(is0, a0hi_ref[...], a1hi_ref[...])
    agg = jnp.concatenate([alo, ahi], axis=1)
    out = agg + x_ref[...]
    h = jnp.dot(out, w1t_ref[0], preferred_element_type=jnp.float32)
    m = jnp.mean(h, axis=0, keepdims=True)
    d = h - m
    v = jnp.mean(d * d, axis=0, keepdims=True)
    hn = d / jnp.sqrt(v + 1e-5) * g_ref[0] + b_ref[0]
    hr = jnp.maximum(hn, 0.0)
    y = jnp.dot(hr, w2t_ref[0], preferred_element_type=jnp.float32)

    @pl.when(c == 0)
    def _():
        accv[...] = y

    @pl.when(c == 1)
    def _():
        t = accv[...] + y
        if last:
            out_ref[...] = t + dep_ref[0, 0]
        else:
            out_ref[...] = jnp.where(t > 0, t, 0.01 * t)


def _layer_call(layer, last, x, a0lo, a0hi, a1lo, a1hi, W1T, g3, b3, W2T,
                dep):
    return pl.pallas_call(
        functools.partial(_layer_body, last),
        grid=(2,),
        in_specs=[
            pl.BlockSpec((N, D), lambda c: (0, 0)),
            pl.BlockSpec((N, H), lambda c: (0, 0)),
            pl.BlockSpec((N, H), lambda c: (0, 0)),
            pl.BlockSpec((N, H), lambda c: (0, 0)),
            pl.BlockSpec((N, H), lambda c: (0, 0)),
            pl.BlockSpec((1, D, 2 * D), lambda c: (2 * layer + c, 0, 0)),
            pl.BlockSpec((1, 1, 2 * D), lambda c: (2 * layer + c, 0, 0)),
            pl.BlockSpec((1, 1, 2 * D), lambda c: (2 * layer + c, 0, 0)),
            pl.BlockSpec((1, 2 * D, D), lambda c: (2 * layer + c, 0, 0)),
            pl.BlockSpec(memory_space=pltpu.SMEM),
        ],
        out_specs=pl.BlockSpec((N, D), lambda c: (0, 0)),
        out_shape=jax.ShapeDtypeStruct((N, D), jnp.float32),
        scratch_shapes=[pltpu.VMEM((N, D), jnp.float32)],
        compiler_params=pltpu.CompilerParams(
            dimension_semantics=("arbitrary",)),
    )(x, a0lo, a0hi, a1lo, a1hi, W1T, g3, b3, W2T, dep)


def kernel(x_hex, ei_flat, ea_flat, lengths, We, W1, gamma, beta, W2):
    ei1d = ei_flat.astype(jnp.int32).reshape(2 * DST_OFF)
    ea3 = ea_flat.reshape(1, 2 * E2, 16)

    WeT = jnp.transpose(We, (0, 2, 1))
    W1T = jnp.transpose(W1, (0, 2, 1))
    W2T = jnp.transpose(W2, (0, 2, 1))
    g3 = gamma.reshape(4, 1, 2 * D)
    b3 = beta.reshape(4, 1, 2 * D)
    dep = (lengths[0] + lengths[1] - 2 * E2).astype(jnp.float32).reshape(1, 1)

    eattrs = [_eattr_call(c, ea3, WeT) for c in range(4)]

    x = x_hex
    for layer in range(2):
        e0lo, e0hi = eattrs[2 * layer + 0]
        e1lo, e1hi = eattrs[2 * layer + 1]
        a0lo, a0hi, a1lo, a1hi = _edge_call(ei1d, x, e0lo, e0hi, e1lo, e1hi)
        x = _layer_call(layer, layer == 1, x, a0lo, a0hi, a1lo, a1hi,
                        W1T, g3, b3, W2T, dep)
    return x


# speedup vs baseline: 3.9416x; 1.1639x over previous
import functools

import jax
import jax.numpy as jnp
from jax import lax
from jax.experimental import pallas as pl
from jax.experimental.pallas import tpu as pltpu
from jax.experimental.pallas import tpu_sc as plsc

N = 10000
D = 128
H = 64
E2 = 160000
EB = 128
NC = 2
NS = 16
TILE_E = E2 // NS
NBLK = TILE_E // EB
REM = TILE_E % EB
DST_OFF = 2 * E2
ACC_R = 5120
P1 = 5104
DUMMY = P1
NPH = 2


def _eattr_body(ea_ref, wet_ref, lo_ref, hi_ref):
    r = jnp.dot(ea_ref[0], wet_ref[0], preferred_element_type=jnp.float32)
    lo_ref[...] = r[:, :H]
    hi_ref[...] = r[:, H:]


def _eattr_call(conv, ea3, WeT):
    Be = 8000
    nb = E2 // Be
    link_off = (conv % 2) * nb
    return pl.pallas_call(
        _eattr_body,
        grid=(nb,),
        in_specs=[
            pl.BlockSpec((1, Be, 16), lambda e: (0, link_off + e, 0)),
            pl.BlockSpec((1, 16, D), lambda e: (conv, 0, 0)),
        ],
        out_specs=[
            pl.BlockSpec((Be, H), lambda e: (e, 0)),
            pl.BlockSpec((Be, H), lambda e: (e, 0)),
        ],
        out_shape=[
            jax.ShapeDtypeStruct((E2, H), jnp.float32),
            jax.ShapeDtypeStruct((E2, H), jnp.float32),
        ],
        compiler_params=pltpu.CompilerParams(
            dimension_semantics=("arbitrary",)),
    )(ea3, WeT)


def _edge_body(ei_hbm, x_hbm, e0lo, e0hi, e1lo, e1hi,
               a0lo, a0hi, a1lo, a1hi,
               idx_s0, idx_s1, idx_d0, idx_d1, xj0, xj1, ea0, ea1, ov0, ov1,
               acc, g0, g1, ss):
    cid = lax.axis_index("c")
    sid = lax.axis_index("s")
    fin = xj0
    res = ea0

    def _stripe(nrows, fn):
        full = nrows // 128
        rem = nrows % 128
        for q in range((full + NS - 1) // NS):
            cix = q * NS + sid

            @pl.when(cix < full)
            def _():
                fn(pl.multiple_of(cix * 128, 128), 128)
        if rem:

            @pl.when(sid == full % NS)
            def _():
                fn(full * 128, rem)

    def _zero_ov():
        def _zrow(r, c):
            for k in range(8):
                ov0[r, pl.ds(k * 16, 16)] = jnp.zeros((16,), jnp.float32)
            return c
        lax.fori_loop(0, EB, _zrow, 0)

    def _compute(xjr, ear, ovr, nrows, xoff, unroll=1):
        def _one(r):
            for k in range(4):
                xv = xjr[r, pl.ds(xoff + k * 16, 16)]
                ev = ear[r, pl.ds(k * 16, 16)]
                m = jnp.maximum(xv + ev, 0.0) + 1e-7
                ex = jnp.exp(m)
                ovr[r, pl.ds(H + k * 16, 16)] = ex
                ovr[r, pl.ds(k * 16, 16)] = m * ex

        def _row(r, c2):
            for u in range(unroll):
                _one(unroll * r + u)
            return c2
        lax.fori_loop(0, nrows // unroll, _row, 0)

    def _remap(idxr, nrows, pbase, pend):
        def _rm(t, c2):
            dv = idxr[pl.ds(t * 16, 16)]
            ok = (dv >= pbase) & (dv < pend)
            idxr[pl.ds(t * 16, 16)] = jnp.where(
                ok, dv - pbase, jnp.full((16,), DUMMY, jnp.int32))
            return c2
        lax.fori_loop(0, nrows // 16, _rm, 0)

    def _scan(link_off, elo_hbm, ehi_hbm, pbase, pend):
        ebase = jnp.where(sid == NS - 1, 15 * 9984, sid * 9984)
        nblk = jnp.where(sid == NS - 1, 80, 78)

        def _issue(b, idx_s, xj, ea, g):
            bi = pl.multiple_of(link_off + ebase + b * EB, 8)
            be = pl.multiple_of(ebase + b * EB, 8)
            pltpu.sync_copy(ei_hbm.at[pl.ds(bi, EB)], idx_s)
            pltpu.async_copy(x_hbm.at[idx_s], xj, g)

            @pl.when(cid == 0)
            def _():
                pltpu.async_copy(elo_hbm.at[pl.ds(be, EB)], ea, g)

            @pl.when(cid == 1)
            def _():
                pltpu.async_copy(ehi_hbm.at[pl.ds(be, EB)], ea, g)

        def _wait_in(xj, ea, gsem):
            pltpu.make_async_copy(x_hbm.at[pl.ds(0, EB)], xj, gsem).wait()
            pltpu.make_async_copy(elo_hbm.at[pl.ds(0, EB)], ea, gsem).wait()

        def _wait_sc(ov):
            pltpu.make_async_copy(ov, acc.at[pl.ds(0, EB)], ss).wait()

        def _load_d(b, idx_d):
            bi = pl.multiple_of(link_off + ebase + b * EB, 8)
            pltpu.sync_copy(ei_hbm.at[pl.ds(DST_OFF + bi, EB)], idx_d)

        def _comp(xj, ea, ov):
            @pl.when(cid == 0)
            def _():
                _compute(xj, ea, ov, EB, 0, unroll=2)

            @pl.when(cid == 1)
            def _():
                _compute(xj, ea, ov, EB, H, unroll=2)

        def _body(j, c):
            b = j - 1
            par = lax.rem(j, 2)

            @pl.when((j < nblk) & (par == 0))
            def _():
                _issue(j, idx_s0, xj0, ea0, g0)

            @pl.when((j < nblk) & (par == 1))
            def _():
                _issue(j, idx_s1, xj1, ea1, g1)

            @pl.when(j > 0)
            def _():
                @pl.when(j > 1)
                def _():
                    _wait_sc(ov0)

                _load_d(b, idx_d0)
                _remap(idx_d0, EB, pbase, pend)

                @pl.when(par == 1)
                def _():
                    _wait_in(xj0, ea0, g0)
                    _comp(xj0, ea0, ov0)

                @pl.when(par == 0)
                def _():
                    _wait_in(xj1, ea1, g1)
                    _comp(xj1, ea1, ov0)

                pltpu.async_copy(ov0, acc.at[idx_d0], ss, add=True)
            return c
        lax.fori_loop(0, nblk + 1, _body, 0)
        _wait_sc(ov0)

    def _mk_fin(lo_hbm, hi_hbm, obase):
        def f(r0, rows):
            pltpu.sync_copy(acc.at[pl.ds(r0, rows)], fin.at[pl.ds(0, rows)])

            def _frow(r, c):
                for k in range(4):
                    w = fin[r, pl.ds(k * 16, 16)]
                    s = fin[r, pl.ds(H + k * 16, 16)]
                    res[r, pl.ds(k * 16, 16)] = w / (s + 1e-16)
                return c
            lax.fori_loop(0, rows, _frow, 0)

            ro = pl.multiple_of(obase + r0, 8)

            @pl.when(cid == 0)
            def _():
                pltpu.sync_copy(res.at[pl.ds(0, rows)],
                                lo_hbm.at[pl.ds(ro, rows)])

            @pl.when(cid == 1)
            def _():
                pltpu.sync_copy(res.at[pl.ds(0, rows)],
                                hi_hbm.at[pl.ds(ro, rows)])
        return f

    convs = (
        (0, e0lo, e0hi, a0lo, a0hi),
        (E2, e1lo, e1hi, a1lo, a1hi),
    )
    for link_off, elo, ehi, alo, ahi in convs:
        for pbase in (0, P1):
            pend = min(pbase + P1, N)
            nrows = pend - pbase
            _zero_ov()
            _stripe(nrows, lambda r0, rows: pltpu.sync_copy(
                ov0.at[pl.ds(0, rows)], acc.at[pl.ds(r0, rows)]))
            plsc.subcore_barrier()
            _scan(link_off, elo, ehi, pbase, pend)
            plsc.subcore_barrier()
            _stripe(nrows, _mk_fin(alo, ahi, pbase))
            plsc.subcore_barrier()


def _edge_call(ei1d, x, e0lo, e0hi, e1lo, e1hi):
    mesh = plsc.VectorSubcoreMesh(
        core_axis_name="c", subcore_axis_name="s",
        num_cores=NC, num_subcores=NS)
    return pl.kernel(
        _edge_body,
        name="edge_pass",
        out_type=[jax.ShapeDtypeStruct((N, H), jnp.float32)
                  for _ in range(4)],
        mesh=mesh,
        scratch_types=[
            pltpu.VMEM((EB,), jnp.int32),
            pltpu.VMEM((EB,), jnp.int32),
            pltpu.VMEM((EB,), jnp.int32),
            pltpu.VMEM((EB,), jnp.int32),
            pltpu.VMEM((EB, D), jnp.float32),
            pltpu.VMEM((EB, D), jnp.float32),
            pltpu.VMEM((EB, H), jnp.float32),
            pltpu.VMEM((EB, H), jnp.float32),
            pltpu.VMEM((EB, D), jnp.float32),
            pltpu.VMEM((EB, D), jnp.float32),
            pltpu.VMEM_SHARED((ACC_R, D), jnp.float32),
            pltpu.SemaphoreType.DMA,
            pltpu.SemaphoreType.DMA,
            pltpu.SemaphoreType.DMA,
        ],
    )(ei1d, x, e0lo, e0hi, e1lo, e1hi)


def _layer_body(last, x_ref, a0lo_ref, a0hi_ref, a1lo_ref, a1hi_ref,
                w1t_ref, g_ref, b_ref, w2t_ref, dep_ref, out_ref, accv):
    c = pl.program_id(0)
    is0 = (c == 0)
    alo = jnp.where(is0, a0lo_ref[...], a1lo_ref[...])
    ahi = jnp.where(is0, a0hi_ref[...], a1hi_ref[...])
    agg = jnp.concatenate([alo, ahi], axis=1)
    out = agg + x_ref[...]
    h = jnp.dot(out, w1t_ref[0], preferred_element_type=jnp.float32)
    m = jnp.mean(h, axis=0, keepdims=True)
    d = h - m
    v = jnp.mean(d * d, axis=0, keepdims=True)
    hn = d / jnp.sqrt(v + 1e-5) * g_ref[0] + b_ref[0]
    hr = jnp.maximum(hn, 0.0)
    y = jnp.dot(hr, w2t_ref[0], preferred_element_type=jnp.float32)

    @pl.when(c == 0)
    def _():
        accv[...] = y

    @pl.when(c == 1)
    def _():
        t = accv[...] + y
        if last:
            out_ref[...] = t + dep_ref[0, 0]
        else:
            out_ref[...] = jnp.where(t > 0, t, 0.01 * t)


def _layer_call(layer, last, x, a0lo, a0hi, a1lo, a1hi, W1T, g3, b3, W2T,
                dep):
    return pl.pallas_call(
        functools.partial(_layer_body, last),
        grid=(2,),
        in_specs=[
            pl.BlockSpec((N, D), lambda c: (0, 0)),
            pl.BlockSpec((N, H), lambda c: (0, 0)),
            pl.BlockSpec((N, H), lambda c: (0, 0)),
            pl.BlockSpec((N, H), lambda c: (0, 0)),
            pl.BlockSpec((N, H), lambda c: (0, 0)),
            pl.BlockSpec((1, D, 2 * D), lambda c: (2 * layer + c, 0, 0)),
            pl.BlockSpec((1, 1, 2 * D), lambda c: (2 * layer + c, 0, 0)),
            pl.BlockSpec((1, 1, 2 * D), lambda c: (2 * layer + c, 0, 0)),
            pl.BlockSpec((1, 2 * D, D), lambda c: (2 * layer + c, 0, 0)),
            pl.BlockSpec(memory_space=pltpu.SMEM),
        ],
        out_specs=pl.BlockSpec((N, D), lambda c: (0, 0)),
        out_shape=jax.ShapeDtypeStruct((N, D), jnp.float32),
        scratch_shapes=[pltpu.VMEM((N, D), jnp.float32)],
        compiler_params=pltpu.CompilerParams(
            dimension_semantics=("arbitrary",)),
    )(x, a0lo, a0hi, a1lo, a1hi, W1T, g3, b3, W2T, dep)


def kernel(x_hex, ei_flat, ea_flat, lengths, We, W1, gamma, beta, W2):
    ei1d = ei_flat.astype(jnp.int32).reshape(2 * DST_OFF)
    ea3 = ea_flat.reshape(1, 2 * E2, 16)

    WeT = jnp.transpose(We, (0, 2, 1))
    W1T = jnp.transpose(W1, (0, 2, 1))
    W2T = jnp.transpose(W2, (0, 2, 1))
    g3 = gamma.reshape(4, 1, 2 * D)
    b3 = beta.reshape(4, 1, 2 * D)
    dep = (lengths[0] + lengths[1] - 2 * E2).astype(jnp.float32).reshape(1, 1)

    eattrs = [_eattr_call(c, ea3, WeT) for c in range(4)]

    x = x_hex
    for layer in range(2):
        e0lo, e0hi = eattrs[2 * layer + 0]
        e1lo, e1hi = eattrs[2 * layer + 1]
        a0lo, a0hi, a1lo, a1hi = _edge_call(ei1d, x, e0lo, e0hi, e1lo, e1hi)
        x = _layer_call(layer, layer == 1, x, a0lo, a0hi, a1lo, a1hi,
                        W1T, g3, b3, W2T, dep)
    return x
